# baseline (device time: 128440 ns/iter reference)
import jax
import jax.numpy as jnp
from jax import lax
from jax.experimental import pallas as pl
from jax.experimental.pallas import tpu as pltpu

N_DEV = 4
SQ = 256
SKV = 4096
D_MODEL = 1024
H_PER = 8
DH = 128
NB = 64
SCALE = 0.08838834764831843

NA = 22
NBB = 23
NC = 23


def _body(x_ref, wq_ref, kf_hbm, vf_hbm, wo_ref, out_ref,
          xg, kA, kB, kC, vA, vB, vC, comm, sbuf,
          ag_send, ag_recv, rs_send, rs_recv, ksems, vsems):
    my = lax.axis_index("i")
    left = (my - 1) % N_DEV
    right = (my + 1) % N_DEV

    barrier_sem = pltpu.get_barrier_semaphore()
    for nbr in [left, right]:
        pl.semaphore_signal(
            barrier_sem, inc=1,
            device_id=(nbr,), device_id_type=pl.DeviceIdType.MESH,
        )
    pl.semaphore_wait(barrier_sem, 2)

    xg[my] = x_ref[0].astype(jnp.bfloat16)

    def ag_hop(h):
        o = (my - h) % N_DEV
        rdma = pltpu.make_async_remote_copy(
            src_ref=xg.at[o], dst_ref=xg.at[o],
            send_sem=ag_send.at[h], recv_sem=ag_recv.at[h],
            device_id=(right,), device_id_type=pl.DeviceIdType.MESH,
        )
        rdma.start()
        return rdma

    def rs_hop(s):
        rdma = pltpu.make_async_remote_copy(
            src_ref=sbuf.at[s], dst_ref=comm.at[s],
            send_sem=rs_send.at[s], recv_sem=rs_recv.at[s],
            device_id=(right,), device_id_type=pl.DeviceIdType.MESH,
        )
        rdma.start()
        return rdma

    batches = [my, (my - 1) % N_DEV, (my - 2) % N_DEV, (my + 1) % N_DEV]

    def _family_copies(i, t):
        bi, h = divmod(t, H_PER)
        slot = t % 2
        hg = my * H_PER + h
        b = batches[bi]
        cs = []
        for sf, bufs, sems in ((kf_hbm, (kA, kB, kC), ksems),
                               (vf_hbm, (vA, vB, vC), vsems)):
            bufA, bufB, bufC = bufs
            cs.append(pltpu.make_async_copy(
                sf.at[b, 3 * i, :, hg, :], bufA.at[slot, i],
                sems.at[slot, 0]))
            cs.append(pltpu.make_async_copy(
                sf.at[b, 3 * i + 2, :, hg, :], bufB.at[slot, 2 + i],
                sems.at[slot, 1]))
            cs.append(pltpu.make_async_copy(
                sf.at[b, 3 * i + 1, :, hg, :], bufC.at[slot, 2 + i],
                sems.at[slot, 2]))
        return cs

    def _extra_copies(t):
        bi, h = divmod(t, H_PER)
        slot = t % 2
        hg = my * H_PER + h
        b = batches[bi]
        cs = []
        for sf, bufs, sems in ((kf_hbm, (kA, kB, kC), ksems),
                               (vf_hbm, (vA, vB, vC), vsems)):
            bufA, bufB, bufC = bufs
            cs.append(pltpu.make_async_copy(
                sf.at[b, 63, :, hg, :], bufA.at[slot, 21], sems.at[slot, 0]))
            cs.append(pltpu.make_async_copy(
                sf.at[b, pl.ds(0, 2), :, hg, :], bufB.at[slot, pl.ds(0, 2)],
                sems.at[slot, 1]))
            cs.append(pltpu.make_async_copy(
                sf.at[b, 0, :, hg, :], bufC.at[slot, 0], sems.at[slot, 2]))
            cs.append(pltpu.make_async_copy(
                sf.at[b, 2, :, hg, :], bufC.at[slot, 1], sems.at[slot, 2]))
        for c in cs:
            c.start()
        return cs

    def start_copies(t):
        def go(i, carry):
            for c in _family_copies(i, t):
                c.start()
            return carry
        lax.fori_loop(0, 21, go, 0)
        _extra_copies(t)
        return t

    def wait_copies(t):
        def go(i, carry):
            for c in _family_copies(i, t):
                c.wait()
            return carry
        lax.fori_loop(0, 21, go, 0)
        slot = t % 2
        for sems in (ksems, vsems):
            pltpu.make_async_copy(
                kf_hbm.at[0, 63, :, 0, :], kA.at[slot, 21],
                sems.at[slot, 0]).wait()
            pltpu.make_async_copy(
                kf_hbm.at[0, pl.ds(0, 2), :, 0, :], kB.at[slot, pl.ds(0, 2)],
                sems.at[slot, 1]).wait()
            pltpu.make_async_copy(
                kf_hbm.at[0, 0, :, 0, :], kC.at[slot, 0],
                sems.at[slot, 2]).wait()
            pltpu.make_async_copy(
                kf_hbm.at[0, 2, :, 0, :], kC.at[slot, 1],
                sems.at[slot, 2]).wait()

    n_steps = N_DEV * H_PER
    start_copies(0)

    def head_step(t, xp16):
        _, h = divmod(t, H_PER)
        slot = t % 2
        if t + 1 < n_steps:
            start_copies(t + 1)
        wait_copies(t)
        q = jnp.dot(xp16, wq_ref[:, h * DH:(h + 1) * DH],
                    preferred_element_type=jnp.float32)
        q16 = q.astype(jnp.bfloat16)

        def piece(qrows, kbuf, vbuf, nblk):
            k16 = kbuf[slot].reshape(nblk * 64, DH).astype(jnp.bfloat16)
            s = lax.dot_general(qrows, k16, (((1,), (1,)), ((), ())),
                                preferred_element_type=jnp.float32)
            e = jnp.exp(s * SCALE)
            d = jnp.sum(e, axis=1, keepdims=True)
            v16 = vbuf[slot].reshape(nblk * 64, DH).astype(jnp.bfloat16)
            ctx = jnp.dot(e.astype(jnp.bfloat16), v16,
                          preferred_element_type=jnp.float32)
            return ctx / d

        ctxA = piece(q16[0:128], kA, vA, NA)
        ctxB = piece(q16[128:192], kB, vB, NBB)
        ctxC = piece(q16[192:256], kC, vC, NC)
        ctx = jnp.concatenate(
            [ctxA[0:64], ctxB, ctxC, ctxA[64:128]], axis=0)
        return jnp.dot(ctx.astype(jnp.bfloat16),
                       wo_ref[h * DH:(h + 1) * DH, :],
                       preferred_element_type=jnp.float32)

    def batch_partial(bi):
        x16 = xg[batches[bi]]
        xp16 = jnp.concatenate(
            [x16[0:64], x16[192:256], x16[64:128], x16[128:192]], axis=0)
        acc = None
        for h in range(H_PER):
            po = head_step(bi * H_PER + h, xp16)
            acc = po if acc is None else acc + po
        return acc

    ag = [None] * (N_DEV - 1)
    rs = [None] * (N_DEV - 1)

    ag[0] = ag_hop(0)
    p_own = batch_partial(0)

    ag[0].wait_recv()
    ag[1] = ag_hop(1)
    p1 = batch_partial(1)
    sbuf[0] = p1.astype(jnp.bfloat16)
    rs[0] = rs_hop(0)

    ag[1].wait_recv()
    ag[2] = ag_hop(2)
    p2 = batch_partial(2)
    rs[0].wait_recv()
    sbuf[1] = (comm[0].astype(jnp.float32) + p2).astype(jnp.bfloat16)
    rs[1] = rs_hop(1)

    ag[2].wait_recv()
    p3 = batch_partial(3)
    rs[1].wait_recv()
    sbuf[2] = (comm[1].astype(jnp.float32) + p3).astype(jnp.bfloat16)
    rs[2] = rs_hop(2)

    rs[2].wait_recv()
    out_ref[0] = comm[2].astype(jnp.float32) + p_own

    for r in ag + rs:
        r.wait_send()


def kernel(x, Wq, K_ext, V_ext, Wo):
    kf = K_ext.reshape(N_DEV, NB, 64, 32, DH)
    vf = V_ext.reshape(N_DEV, NB, 64, 32, DH)
    return pl.pallas_call(
        _body,
        out_shape=jax.ShapeDtypeStruct((1, SQ, D_MODEL), jnp.float32),
        in_specs=[
            pl.BlockSpec(memory_space=pltpu.VMEM),
            pl.BlockSpec(memory_space=pltpu.VMEM),
            pl.BlockSpec(memory_space=pl.ANY),
            pl.BlockSpec(memory_space=pl.ANY),
            pl.BlockSpec(memory_space=pltpu.VMEM),
        ],
        out_specs=pl.BlockSpec(memory_space=pltpu.VMEM),
        scratch_shapes=[
            pltpu.VMEM((N_DEV, SQ, D_MODEL), jnp.bfloat16),
            pltpu.VMEM((2, NA, 64, DH), jnp.float32),
            pltpu.VMEM((2, NBB, 64, DH), jnp.float32),
            pltpu.VMEM((2, NC, 64, DH), jnp.float32),
            pltpu.VMEM((2, NA, 64, DH), jnp.float32),
            pltpu.VMEM((2, NBB, 64, DH), jnp.float32),
            pltpu.VMEM((2, NC, 64, DH), jnp.float32),
            pltpu.VMEM((N_DEV - 1, SQ, D_MODEL), jnp.bfloat16),
            pltpu.VMEM((N_DEV - 1, SQ, D_MODEL), jnp.bfloat16),
            pltpu.SemaphoreType.DMA((N_DEV - 1,)),
            pltpu.SemaphoreType.DMA((N_DEV - 1,)),
            pltpu.SemaphoreType.DMA((N_DEV - 1,)),
            pltpu.SemaphoreType.DMA((N_DEV - 1,)),
            pltpu.SemaphoreType.DMA((2, 3)),
            pltpu.SemaphoreType.DMA((2, 3)),
        ],
        compiler_params=pltpu.CompilerParams(collective_id=0),
    )(x, Wq.astype(jnp.bfloat16), kf, vf, Wo.astype(jnp.bfloat16))
